# Initial kernel scaffold; baseline (speedup 1.0000x reference)
import jax
import jax.numpy as jnp
from jax.experimental import pallas as pl


def kernel(x):
    raise NotImplementedError("write your pallas kernel here")


# speedup vs baseline: 3.0820x; 3.0820x over previous
import jax
import jax.numpy as jnp
from jax.experimental import pallas as pl
from jax.experimental.pallas import tpu as pltpu

B, N, F = 4, 8192, 2048
S = 256


def _body(x_ref, o_ref, carry_ref):
    s = pl.program_id(1)

    @pl.when(s == 0)
    def _():
        carry_ref[...] = jnp.zeros_like(carry_ref)

    x = x_ref[0]
    r = jax.lax.broadcasted_iota(jnp.int32, (S, S), 0)
    c = jax.lax.broadcasted_iota(jnp.int32, (S, S), 1)
    tri = (c <= r).astype(jnp.float32)
    acc = jnp.dot(tri, x, preferred_element_type=jnp.float32)
    acc = acc + carry_ref[...]
    o_ref[0] = acc
    carry_ref[...] = acc[S - 1 : S, :]


def kernel(x):
    return pl.pallas_call(
        _body,
        grid=(B, N // S),
        in_specs=[pl.BlockSpec((1, S, F), lambda b, s: (b, s, 0))],
        out_specs=pl.BlockSpec((1, S, F), lambda b, s: (b, s, 0)),
        out_shape=jax.ShapeDtypeStruct((B, N, F), jnp.float32),
        scratch_shapes=[pltpu.VMEM((1, F), jnp.float32)],
    )(x)
